# baseline (speedup 1.0000x reference)
import functools

import jax
import jax.numpy as jnp
from jax import lax
from jax.experimental import pallas as pl
from jax.experimental.pallas import tpu as pltpu
from jax.experimental.pallas import tpu_sc as plsc

N = 10000
E = 320000
D = 128

NC = 2
NS = 16
NW = NC * NS
EW = E // NW
K = 80
M = EW // K
S = 25
NSC = M // S
RB = 624
REM = N - NS * RB

@functools.cache
def _sc_mesh():
    return plsc.VectorSubcoreMesh(
        core_axis_name="c", subcore_axis_name="s", num_cores=NC, num_subcores=NS)


_DEPTH = 8


def _deg_body(dst_hbm, ones_hbm, zeros_hbm, out_hbm, idx_v, ones_v, acc_sh, sem):
    c = lax.axis_index("c")
    s = lax.axis_index("s")
    wid = s * NC + c

    pltpu.sync_copy(ones_hbm, ones_v)
    pltpu.sync_copy(zeros_hbm, acc_sh.at[pl.ds(s * RB, RB)])

    @pl.when(s == NS - 1)
    def _():
        pltpu.sync_copy(zeros_hbm.at[pl.ds(0, REM)], acc_sh.at[pl.ds(NS * RB, REM)])
    plsc.subcore_barrier()

    for sc in range(NSC):
        pltpu.sync_copy(dst_hbm.at[wid, sc], idx_v)

        def step(i, _):
            pltpu.async_copy(ones_v, acc_sh.at[idx_v.at[i]], sem, add=True)

            @pl.when(i >= _DEPTH)
            def _():
                pltpu.make_async_copy(ones_v, acc_sh.at[idx_v.at[0]], sem).wait()
            return 0
        lax.fori_loop(0, S, step, 0)
        for _ in range(_DEPTH):
            pltpu.make_async_copy(ones_v, acc_sh.at[idx_v.at[0]], sem).wait()
    plsc.subcore_barrier()

    pltpu.sync_copy(acc_sh.at[pl.ds(s * RB, RB)], out_hbm.at[c, pl.ds(s * RB, RB)])

    @pl.when(s == NS - 1)
    def _():
        pltpu.sync_copy(acc_sh.at[pl.ds(NS * RB, REM)],
                        out_hbm.at[c, pl.ds(NS * RB, REM)])


@functools.cache
def _deg_call():
    return pl.kernel(
        _deg_body,
        out_type=jax.ShapeDtypeStruct((NC, N, D), jnp.float32),
        mesh=_sc_mesh(),
        scratch_types=[
            pltpu.VMEM((S, K), jnp.int32),
            pltpu.VMEM((K, D), jnp.float32),
            pltpu.VMEM_SHARED((N, D), jnp.float32),
            pltpu.SemaphoreType.DMA,
        ],
    )


def _prop_body(g_hbm, src_hbm, dst_hbm, out_hbm,
               idx_s, idx_d, rows0, rows1, acc_sh, gs0, gs1, ss0, ss1):
    c = lax.axis_index("c")
    s = lax.axis_index("s")
    wid = s * NC + c
    r0 = s * RB

    pltpu.sync_copy(g_hbm.at[pl.ds(r0, RB)], acc_sh.at[pl.ds(r0, RB)])

    @pl.when(s == NS - 1)
    def _():
        pltpu.sync_copy(g_hbm.at[pl.ds(NS * RB, REM)], acc_sh.at[pl.ds(NS * RB, REM)])
    plsc.subcore_barrier()

    def gather(i, buf, sem):
        pltpu.async_copy(g_hbm.at[idx_s.at[i]], buf, sem)

    def gwait(i, buf, sem):
        pltpu.make_async_copy(g_hbm.at[idx_s.at[i]], buf, sem).wait()

    def scat(i, buf, sem):
        pltpu.async_copy(buf, acc_sh.at[idx_d.at[i]], sem, add=True)

    def swait(buf, sem):
        pltpu.make_async_copy(buf, acc_sh.at[idx_d.at[0]], sem).wait()

    for sc in range(NSC):
        pltpu.sync_copy(src_hbm.at[wid, sc], idx_s)
        pltpu.sync_copy(dst_hbm.at[wid, sc], idx_d)
        gather(0, rows0, gs0)
        gwait(0, rows0, gs0)
        scat(0, rows0, ss0)
        gather(1, rows1, gs1)
        gwait(1, rows1, gs1)
        scat(1, rows1, ss1)
        swait(rows0, ss0)
        gather(2, rows0, gs0)

        def pair(k, _):
            i = 2 * k + 2
            gwait(i, rows0, gs0)
            scat(i, rows0, ss0)
            swait(rows1, ss1)
            gather(i + 1, rows1, gs1)
            gwait(i + 1, rows1, gs1)
            scat(i + 1, rows1, ss1)
            swait(rows0, ss0)
            gather(i + 2, rows0, gs0)
            return 0
        lax.fori_loop(0, (S - 3) // 2, pair, 0)
        gwait(S - 1, rows0, gs0)
        scat(S - 1, rows0, ss0)
        swait(rows1, ss1)
        swait(rows0, ss0)
    plsc.subcore_barrier()

    pltpu.sync_copy(acc_sh.at[pl.ds(r0, RB)], out_hbm.at[c, pl.ds(r0, RB)])

    @pl.when(s == NS - 1)
    def _():
        pltpu.sync_copy(acc_sh.at[pl.ds(NS * RB, REM)],
                        out_hbm.at[c, pl.ds(NS * RB, REM)])


@functools.cache
def _prop_call():
    return pl.kernel(
        _prop_body,
        out_type=jax.ShapeDtypeStruct((NC, N, D), jnp.float32),
        mesh=_sc_mesh(),
        scratch_types=[
            pltpu.VMEM((S, K), jnp.int32),
            pltpu.VMEM((S, K), jnp.int32),
            pltpu.VMEM((K, D), jnp.float32),
            pltpu.VMEM((K, D), jnp.float32),
            pltpu.VMEM_SHARED((N, D), jnp.float32),
            pltpu.SemaphoreType.DMA,
            pltpu.SemaphoreType.DMA,
            pltpu.SemaphoreType.DMA,
            pltpu.SemaphoreType.DMA,
        ],
    )


_BB = 1000


def _pre_body(x_ref, w_ref, degp_ref, g_ref, dinv_ref):
    degp = degp_ref[...]
    deg = 1.0 + degp[0, :, 0] + degp[1, :, 0]
    di = lax.rsqrt(deg)
    xw = jnp.dot(x_ref[...], w_ref[...], preferred_element_type=jnp.float32)
    g_ref[...] = xw * di[:, None]
    dinv_ref[...] = jnp.broadcast_to(di[:, None], (_BB, D))


def _pre_call(x, w1, degp):
    return pl.pallas_call(
        _pre_body,
        grid=(N // _BB,),
        in_specs=[
            pl.BlockSpec((_BB, D), lambda i: (i, 0)),
            pl.BlockSpec((D, D), lambda i: (0, 0)),
            pl.BlockSpec((NC, _BB, D), lambda i: (0, i, 0)),
        ],
        out_specs=[
            pl.BlockSpec((_BB, D), lambda i: (i, 0)),
            pl.BlockSpec((_BB, D), lambda i: (i, 0)),
        ],
        out_shape=[
            jax.ShapeDtypeStruct((N, D), jnp.float32),
            jax.ShapeDtypeStruct((N, D), jnp.float32),
        ],
    )(x, w1, degp)


def _mid_body(acc_ref, g1_ref, dinv_ref, b1_ref, w2_ref, g2_ref):
    di = dinv_ref[...]
    acc = acc_ref[...]
    agg = (acc[0] + acc[1] - g1_ref[...]) * di + b1_ref[...]
    h1 = jnp.maximum(agg, 0.0)
    g2_ref[...] = jnp.dot(h1, w2_ref[...], preferred_element_type=jnp.float32) * di


def _mid_call(acc, g1, dinv, b1, w2):
    return pl.pallas_call(
        _mid_body,
        grid=(N // _BB,),
        in_specs=[
            pl.BlockSpec((NC, _BB, D), lambda i: (0, i, 0)),
            pl.BlockSpec((_BB, D), lambda i: (i, 0)),
            pl.BlockSpec((_BB, D), lambda i: (i, 0)),
            pl.BlockSpec((1, D), lambda i: (0, 0)),
            pl.BlockSpec((D, D), lambda i: (0, 0)),
        ],
        out_specs=pl.BlockSpec((_BB, D), lambda i: (i, 0)),
        out_shape=jax.ShapeDtypeStruct((N, D), jnp.float32),
    )(acc, g1, dinv, b1, w2)


def _post_body(acc_ref, g2_ref, dinv_ref, b2_ref, out_ref):
    acc = acc_ref[...]
    out_ref[...] = (acc[0] + acc[1] - g2_ref[...]) * dinv_ref[...] + b2_ref[...]


def _post_call(acc, g2, dinv, b2):
    return pl.pallas_call(
        _post_body,
        grid=(N // _BB,),
        in_specs=[
            pl.BlockSpec((NC, _BB, D), lambda i: (0, i, 0)),
            pl.BlockSpec((_BB, D), lambda i: (i, 0)),
            pl.BlockSpec((_BB, D), lambda i: (i, 0)),
            pl.BlockSpec((1, D), lambda i: (0, 0)),
        ],
        out_specs=pl.BlockSpec((_BB, D), lambda i: (i, 0)),
        out_shape=jax.ShapeDtypeStruct((N, D), jnp.float32),
    )(acc, g2, dinv, b2)


def kernel(x, edge_index, W1, b1, W2, b2):
    src4 = edge_index[0].reshape(NW, NSC, S, K)
    dst4 = edge_index[1].reshape(NW, NSC, S, K)
    b1r = b1.reshape(1, D)
    b2r = b2.reshape(1, D)

    ones_c = jnp.ones((K, D), jnp.float32)
    zeros_c = jnp.zeros((RB, D), jnp.float32)
    degp = _deg_call()(dst4, ones_c, zeros_c)
    g1, dinv = _pre_call(x, W1, degp)
    acc1 = _prop_call()(g1, src4, dst4)
    g2 = _mid_call(acc1, g1, dinv, b1r, W2)
    acc2 = _prop_call()(g2, src4, dst4)
    return _post_call(acc2, g2, dinv, b2r)


# speedup vs baseline: 1.1757x; 1.1757x over previous
import functools

import jax
import jax.numpy as jnp
from jax import lax
from jax.experimental import pallas as pl
from jax.experimental.pallas import tpu as pltpu
from jax.experimental.pallas import tpu_sc as plsc

N = 10000
E = 320000
D = 128

NC = 2
NS = 16
NW = NC * NS
EW = E // NW
K = 80
M = EW // K
S = 25
NSC = M // S
RB = 624
REM = N - NS * RB

@functools.cache
def _sc_mesh():
    return plsc.VectorSubcoreMesh(
        core_axis_name="c", subcore_axis_name="s", num_cores=NC, num_subcores=NS)


_DEPTH = 8


def _deg_body(dst_hbm, ones_hbm, zeros_hbm, out_hbm, idx_v, ones_v, acc_sh, sem):
    c = lax.axis_index("c")
    s = lax.axis_index("s")
    wid = s * NC + c

    pltpu.sync_copy(ones_hbm, ones_v)
    pltpu.sync_copy(zeros_hbm, acc_sh.at[pl.ds(s * RB, RB)])

    @pl.when(s == NS - 1)
    def _():
        pltpu.sync_copy(zeros_hbm.at[pl.ds(0, REM)], acc_sh.at[pl.ds(NS * RB, REM)])
    plsc.subcore_barrier()

    pltpu.sync_copy(dst_hbm.at[wid], idx_v)

    def step(i, _):
        pltpu.async_copy(ones_v, acc_sh.at[idx_v.at[i]], sem, add=True)

        @pl.when(i >= _DEPTH)
        def _():
            pltpu.make_async_copy(ones_v, acc_sh.at[idx_v.at[0]], sem).wait()
        return 0
    lax.fori_loop(0, M, step, 0)
    for _ in range(_DEPTH):
        pltpu.make_async_copy(ones_v, acc_sh.at[idx_v.at[0]], sem).wait()
    plsc.subcore_barrier()

    pltpu.sync_copy(acc_sh.at[pl.ds(s * RB, RB)], out_hbm.at[c, pl.ds(s * RB, RB)])

    @pl.when(s == NS - 1)
    def _():
        pltpu.sync_copy(acc_sh.at[pl.ds(NS * RB, REM)],
                        out_hbm.at[c, pl.ds(NS * RB, REM)])


@functools.cache
def _deg_call():
    return pl.kernel(
        _deg_body,
        out_type=jax.ShapeDtypeStruct((NC, N, D), jnp.float32),
        mesh=_sc_mesh(),
        scratch_types=[
            pltpu.VMEM((M, K), jnp.int32),
            pltpu.VMEM((K, D), jnp.float32),
            pltpu.VMEM_SHARED((N, D), jnp.float32),
            pltpu.SemaphoreType.DMA,
        ],
    )


def _prop_body(g_hbm, src_hbm, dst_hbm, out_hbm,
               idx_s, idx_d, rows0, rows1, acc_sh, gs0, gs1):
    c = lax.axis_index("c")
    s = lax.axis_index("s")
    wid = s * NC + c
    r0 = s * RB

    pltpu.sync_copy(g_hbm.at[pl.ds(r0, RB)], acc_sh.at[pl.ds(r0, RB)])

    @pl.when(s == NS - 1)
    def _():
        pltpu.sync_copy(g_hbm.at[pl.ds(NS * RB, REM)], acc_sh.at[pl.ds(NS * RB, REM)])
    plsc.subcore_barrier()

    for sc in range(NSC):
        pltpu.sync_copy(src_hbm.at[wid, sc], idx_s)
        pltpu.sync_copy(dst_hbm.at[wid, sc], idx_d)
        pltpu.async_copy(g_hbm.at[idx_s.at[0]], rows0, gs0)

        @pl.loop(0, S - 1, step=2)
        def _(i):
            pltpu.async_copy(g_hbm.at[idx_s.at[i + 1]], rows1, gs1)
            pltpu.make_async_copy(g_hbm.at[idx_s.at[i]], rows0, gs0).wait()
            pltpu.sync_copy(rows0, acc_sh.at[idx_d.at[i]], add=True)
            pltpu.async_copy(g_hbm.at[idx_s.at[i + 2]], rows0, gs0)
            pltpu.make_async_copy(g_hbm.at[idx_s.at[i + 1]], rows1, gs1).wait()
            pltpu.sync_copy(rows1, acc_sh.at[idx_d.at[i + 1]], add=True)

        pltpu.make_async_copy(g_hbm.at[idx_s.at[S - 1]], rows0, gs0).wait()
        pltpu.sync_copy(rows0, acc_sh.at[idx_d.at[S - 1]], add=True)
    plsc.subcore_barrier()

    pltpu.sync_copy(acc_sh.at[pl.ds(r0, RB)], out_hbm.at[c, pl.ds(r0, RB)])

    @pl.when(s == NS - 1)
    def _():
        pltpu.sync_copy(acc_sh.at[pl.ds(NS * RB, REM)],
                        out_hbm.at[c, pl.ds(NS * RB, REM)])


@functools.cache
def _prop_call():
    return pl.kernel(
        _prop_body,
        out_type=jax.ShapeDtypeStruct((NC, N, D), jnp.float32),
        mesh=_sc_mesh(),
        scratch_types=[
            pltpu.VMEM((S, K), jnp.int32),
            pltpu.VMEM((S, K), jnp.int32),
            pltpu.VMEM((K, D), jnp.float32),
            pltpu.VMEM((K, D), jnp.float32),
            pltpu.VMEM_SHARED((N, D), jnp.float32),
            pltpu.SemaphoreType.DMA,
            pltpu.SemaphoreType.DMA,
        ],
    )


_BB = 1000


def _pre_body(x_ref, w_ref, degp_ref, g_ref, dinv_ref):
    degp = degp_ref[...]
    deg = 1.0 + degp[0, :, 0] + degp[1, :, 0]
    di = lax.rsqrt(deg)
    xw = jnp.dot(x_ref[...], w_ref[...], preferred_element_type=jnp.float32)
    g_ref[...] = xw * di[:, None]
    dinv_ref[...] = jnp.broadcast_to(di[:, None], (_BB, D))


def _pre_call(x, w1, degp):
    return pl.pallas_call(
        _pre_body,
        grid=(N // _BB,),
        in_specs=[
            pl.BlockSpec((_BB, D), lambda i: (i, 0)),
            pl.BlockSpec((D, D), lambda i: (0, 0)),
            pl.BlockSpec((NC, _BB, D), lambda i: (0, i, 0)),
        ],
        out_specs=[
            pl.BlockSpec((_BB, D), lambda i: (i, 0)),
            pl.BlockSpec((_BB, D), lambda i: (i, 0)),
        ],
        out_shape=[
            jax.ShapeDtypeStruct((N, D), jnp.float32),
            jax.ShapeDtypeStruct((N, D), jnp.float32),
        ],
    )(x, w1, degp)


def _mid_body(acc_ref, g1_ref, dinv_ref, b1_ref, w2_ref, g2_ref):
    di = dinv_ref[...]
    acc = acc_ref[...]
    agg = (acc[0] + acc[1] - g1_ref[...]) * di + b1_ref[...]
    h1 = jnp.maximum(agg, 0.0)
    g2_ref[...] = jnp.dot(h1, w2_ref[...], preferred_element_type=jnp.float32) * di


def _mid_call(acc, g1, dinv, b1, w2):
    return pl.pallas_call(
        _mid_body,
        grid=(N // _BB,),
        in_specs=[
            pl.BlockSpec((NC, _BB, D), lambda i: (0, i, 0)),
            pl.BlockSpec((_BB, D), lambda i: (i, 0)),
            pl.BlockSpec((_BB, D), lambda i: (i, 0)),
            pl.BlockSpec((1, D), lambda i: (0, 0)),
            pl.BlockSpec((D, D), lambda i: (0, 0)),
        ],
        out_specs=pl.BlockSpec((_BB, D), lambda i: (i, 0)),
        out_shape=jax.ShapeDtypeStruct((N, D), jnp.float32),
    )(acc, g1, dinv, b1, w2)


def _post_body(acc_ref, g2_ref, dinv_ref, b2_ref, out_ref):
    acc = acc_ref[...]
    out_ref[...] = (acc[0] + acc[1] - g2_ref[...]) * dinv_ref[...] + b2_ref[...]


def _post_call(acc, g2, dinv, b2):
    return pl.pallas_call(
        _post_body,
        grid=(N // _BB,),
        in_specs=[
            pl.BlockSpec((NC, _BB, D), lambda i: (0, i, 0)),
            pl.BlockSpec((_BB, D), lambda i: (i, 0)),
            pl.BlockSpec((_BB, D), lambda i: (i, 0)),
            pl.BlockSpec((1, D), lambda i: (0, 0)),
        ],
        out_specs=pl.BlockSpec((_BB, D), lambda i: (i, 0)),
        out_shape=jax.ShapeDtypeStruct((N, D), jnp.float32),
    )(acc, g2, dinv, b2)


def kernel(x, edge_index, W1, b1, W2, b2):
    src4 = edge_index[0].reshape(NW, NSC, S, K)
    dst4 = edge_index[1].reshape(NW, NSC, S, K)
    dst3 = edge_index[1].reshape(NW, M, K)
    b1r = b1.reshape(1, D)
    b2r = b2.reshape(1, D)

    ones_c = jnp.ones((K, D), jnp.float32)
    zeros_c = jnp.zeros((RB, D), jnp.float32)
    degp = _deg_call()(dst3, ones_c, zeros_c)
    g1, dinv = _pre_call(x, W1, degp)
    acc1 = _prop_call()(g1, src4, dst4)
    g2 = _mid_call(acc1, g1, dinv, b1r, W2)
    acc2 = _prop_call()(g2, src4, dst4)
    return _post_call(acc2, g2, dinv, b2r)
